# initial kernel scaffold (unmeasured)
import numpy as np
import jax
import jax.numpy as jnp
from jax import lax
from jax.experimental import pallas as pl
from jax.experimental.pallas import tpu as pltpu

N_DEV = 32
Dh = 64


def kernel(x, Wq, Wk, Wv, Wo):
    B_loc, Sq, D = x.shape
    _, HD = Wq.shape
    H_loc = HD // Dh
    T = B_loc * Sq
    f32 = jnp.float32
    bf16 = jnp.bfloat16

    inv = 1.0 / (10000.0 ** (np.arange(0, Dh, 2) / Dh))
    pos = np.arange(Sq)[:, None] * inv[None, :]
    cos = np.repeat(np.cos(pos), 2, axis=-1)
    sin = np.repeat(np.sin(pos), 2, axis=-1)
    cosw = jnp.asarray(np.tile(cos, (B_loc, H_loc)), dtype=f32)
    sinw = jnp.asarray(np.tile(sin, (B_loc, H_loc)), dtype=f32)
    P64 = np.zeros((Dh, Dh), np.float32)
    P64[np.arange(1, Dh, 2), np.arange(0, Dh, 2)] = -1.0
    P64[np.arange(0, Dh, 2), np.arange(1, Dh, 2)] = 1.0
    P = jnp.asarray(np.kron(np.eye(H_loc), P64), dtype=bf16)

    def body(x_ref, wq_ref, wk_ref, wv_ref, wo_ref, cosw_ref, sinw_ref,
             p_ref, out_ref,
             xf_ref, ctx_ref, acc_ref, qkv_comm, wo_comm,
             qkv_send, qkv_recv, wo_send, wo_recv, credit_sem):
        me = lax.axis_index("i")
        left = lax.rem(me - 1 + N_DEV, N_DEV)
        right = lax.rem(me + 1, N_DEV)

        bsem = pltpu.get_barrier_semaphore()
        for nbr in (left, right):
            pl.semaphore_signal(bsem, inc=1, device_id=(nbr,),
                                device_id_type=pl.DeviceIdType.MESH)
        pl.semaphore_wait(bsem, 2)

        for b in range(B_loc):
            xf_ref[pl.ds(b * Sq, Sq), :] = x_ref[b].astype(bf16)

        qkv_comm[0, :, 0:HD] = wq_ref[...].astype(bf16)
        qkv_comm[0, :, HD:2 * HD] = wk_ref[...].astype(bf16)
        qkv_comm[0, :, 2 * HD:3 * HD] = wv_ref[...].astype(bf16)
        wo_comm[0] = wo_ref[...].astype(bf16)
        acc_ref[...] = jnp.zeros((T, D), f32)

        def compute_group(qkv, wo):
            xf = xf_ref[...]
            p16 = p_ref[...]
            cw = cosw_ref[...]
            sw = sinw_ref[...]
            q = jnp.dot(xf, qkv[:, 0:HD], preferred_element_type=f32)
            k = jnp.dot(xf, qkv[:, HD:2 * HD], preferred_element_type=f32)
            v = jnp.dot(xf, qkv[:, 2 * HD:3 * HD],
                        preferred_element_type=f32).astype(bf16)
            qr = (q * cw + jnp.dot(q.astype(bf16), p16,
                                   preferred_element_type=f32) * sw).astype(bf16)
            kr = (k * cw + jnp.dot(k.astype(bf16), p16,
                                   preferred_element_type=f32) * sw).astype(bf16)
            for b in range(B_loc):
                for h in range(H_loc):
                    rs = slice(b * Sq, (b + 1) * Sq)
                    cs = slice(h * Dh, (h + 1) * Dh)
                    s = lax.dot_general(qr[rs, cs], kr[rs, cs],
                                        (((1,), (1,)), ((), ())),
                                        preferred_element_type=f32) * 0.125
                    m = jnp.max(s, axis=-1, keepdims=True)
                    w = jnp.exp(s - m)
                    w = (w / jnp.sum(w, axis=-1, keepdims=True)).astype(bf16)
                    ctx_ref[pl.ds(b * Sq, Sq), pl.ds(h * Dh, Dh)] = jnp.dot(
                        w, v[rs, cs], preferred_element_type=f32).astype(bf16)
            acc_ref[...] += jnp.dot(ctx_ref[...], wo,
                                    preferred_element_type=f32)

        compute_group(qkv_comm[0], wo_comm[0])

        def do_hop(slot_s, slot_r, credit_pred, emit_pred):
            rq = pltpu.make_async_remote_copy(
                src_ref=qkv_comm.at[slot_s], dst_ref=qkv_comm.at[slot_r],
                send_sem=qkv_send.at[slot_s], recv_sem=qkv_recv.at[slot_r],
                device_id=(right,), device_id_type=pl.DeviceIdType.MESH)
            ro = pltpu.make_async_remote_copy(
                src_ref=wo_comm.at[slot_s], dst_ref=wo_comm.at[slot_r],
                send_sem=wo_send.at[slot_s], recv_sem=wo_recv.at[slot_r],
                device_id=(right,), device_id_type=pl.DeviceIdType.MESH)

            def start():
                rq.start()
                ro.start()

            if credit_pred is True:
                pl.semaphore_wait(credit_sem, 1)
                start()
            else:
                @pl.when(credit_pred)
                def _():
                    pl.semaphore_wait(credit_sem, 1)
                start()
            rq.wait()
            ro.wait()
            if emit_pred is True:
                pl.semaphore_signal(credit_sem, inc=1, device_id=(left,),
                                    device_id_type=pl.DeviceIdType.MESH)
            elif emit_pred is not False:
                @pl.when(emit_pred)
                def _():
                    pl.semaphore_signal(credit_sem, inc=1, device_id=(left,),
                                        device_id_type=pl.DeviceIdType.MESH)
            compute_group(qkv_comm[slot_r], wo_comm[slot_r])

        def loop_body(it, carry):
            do_hop(0, 1, it >= 1, it >= 1)
            do_hop(1, 0, it >= 1, True)
            return carry

        lax.fori_loop(0, 15, loop_body, 0)
        do_hop(0, 1, True, False)

        for b in range(B_loc):
            out_ref[b] = acc_ref[pl.ds(b * Sq, Sq), :]

    return pl.pallas_call(
        body,
        out_shape=jax.ShapeDtypeStruct((B_loc, Sq, D), f32),
        in_specs=[pl.BlockSpec(memory_space=pltpu.VMEM)] * 8,
        out_specs=pl.BlockSpec(memory_space=pltpu.VMEM),
        scratch_shapes=[
            pltpu.VMEM((T, D), bf16),
            pltpu.VMEM((T, HD), bf16),
            pltpu.VMEM((T, D), f32),
            pltpu.VMEM((2, D, 3 * HD), bf16),
            pltpu.VMEM((2, HD, D), bf16),
            pltpu.SemaphoreType.DMA((2,)),
            pltpu.SemaphoreType.DMA((2,)),
            pltpu.SemaphoreType.DMA((2,)),
            pltpu.SemaphoreType.DMA((2,)),
            pltpu.SemaphoreType.REGULAR,
        ],
        compiler_params=pltpu.CompilerParams(collective_id=0),
    )(x, Wq, Wk, Wv, Wo, cosw, sinw, P)


# baseline (device time: 700235 ns/iter reference)
import numpy as np
import jax
import jax.numpy as jnp
from jax import lax
from jax.experimental import pallas as pl
from jax.experimental.pallas import tpu as pltpu

N_DEV = 32
Dh = 64


def kernel(x, Wq, Wk, Wv, Wo):
    B_loc, Sq, D = x.shape
    _, HD = Wq.shape
    H_loc = HD // Dh
    T = B_loc * Sq
    f32 = jnp.float32
    bf16 = jnp.bfloat16

    inv = 1.0 / (10000.0 ** (np.arange(0, Dh, 2) / Dh))
    pos = np.arange(Sq)[:, None] * inv[None, :]
    cos = np.repeat(np.cos(pos), 2, axis=-1)
    sin = np.repeat(np.sin(pos), 2, axis=-1)
    cosw = jnp.asarray(np.tile(cos, (B_loc, H_loc)), dtype=f32)
    sinw = jnp.asarray(np.tile(sin, (B_loc, H_loc)), dtype=f32)
    P64 = np.zeros((Dh, Dh), np.float32)
    P64[np.arange(1, Dh, 2), np.arange(0, Dh, 2)] = -1.0
    P64[np.arange(0, Dh, 2), np.arange(1, Dh, 2)] = 1.0
    P = jnp.asarray(np.kron(np.eye(H_loc), P64), dtype=bf16)

    def body(x_ref, wq_ref, wk_ref, wv_ref, wo_ref, cosw_ref, sinw_ref,
             p_ref, out_ref,
             xf_ref, ctx_ref, acc_ref, comm, send_sems, recv_sems):
        me = lax.axis_index("i")
        left = lax.rem(me - 1 + N_DEV, N_DEV)
        right = lax.rem(me + 1, N_DEV)

        bsem = pltpu.get_barrier_semaphore()
        for nbr in (left, right):
            pl.semaphore_signal(bsem, inc=1, device_id=(nbr,),
                                device_id_type=pl.DeviceIdType.MESH)
        pl.semaphore_wait(bsem, 2)

        for b in range(B_loc):
            xf_ref[pl.ds(b * Sq, Sq), :] = x_ref[b].astype(bf16)

        comm[0, 0:D, 0:HD] = wq_ref[...].astype(bf16)
        comm[0, 0:D, HD:2 * HD] = wk_ref[...].astype(bf16)
        comm[0, 0:D, 2 * HD:3 * HD] = wv_ref[...].astype(bf16)
        comm[0, D:D + HD, :] = wo_ref[...].astype(bf16)
        acc_ref[...] = jnp.zeros((T, D), f32)

        def compute_group(slot):
            qkv = comm[slot, 0:D, :]
            wo = comm[slot, D:D + HD, :]
            xf = xf_ref[...]
            p16 = p_ref[...]
            cw = cosw_ref[...]
            sw = sinw_ref[...]
            q = jnp.dot(xf, qkv[:, 0:HD], preferred_element_type=f32)
            k = jnp.dot(xf, qkv[:, HD:2 * HD], preferred_element_type=f32)
            v = jnp.dot(xf, qkv[:, 2 * HD:3 * HD],
                        preferred_element_type=f32).astype(bf16)
            qr = (q * cw + jnp.dot(q.astype(bf16), p16,
                                   preferred_element_type=f32) * sw).astype(bf16)
            kr = (k * cw + jnp.dot(k.astype(bf16), p16,
                                   preferred_element_type=f32) * sw).astype(bf16)
            for b in range(B_loc):
                for h in range(H_loc):
                    rs = slice(b * Sq, (b + 1) * Sq)
                    cs = slice(h * Dh, (h + 1) * Dh)
                    s = lax.dot_general(qr[rs, cs], kr[rs, cs],
                                        (((1,), (1,)), ((), ())),
                                        preferred_element_type=f32) * 0.125
                    m = jnp.max(s, axis=-1, keepdims=True)
                    w = jnp.exp(s - m)
                    w = (w / jnp.sum(w, axis=-1, keepdims=True)).astype(bf16)
                    ctx_ref[pl.ds(b * Sq, Sq), pl.ds(h * Dh, Dh)] = jnp.dot(
                        w, v[rs, cs], preferred_element_type=f32).astype(bf16)
            acc_ref[...] += jnp.dot(ctx_ref[...], wo,
                                    preferred_element_type=f32)

        compute_group(0)

        def do_hop(slot_s, slot_r):
            rdma = pltpu.make_async_remote_copy(
                src_ref=comm.at[slot_s], dst_ref=comm.at[slot_r],
                send_sem=send_sems.at[slot_s], recv_sem=recv_sems.at[slot_r],
                device_id=(right,), device_id_type=pl.DeviceIdType.MESH)
            rdma.start()
            rdma.wait()
            compute_group(slot_r)

        def loop_body(it, carry):
            do_hop(0, 1)
            do_hop(1, 0)
            return carry

        lax.fori_loop(0, 15, loop_body, 0)
        do_hop(0, 1)

        for b in range(B_loc):
            out_ref[b] = acc_ref[pl.ds(b * Sq, Sq), :]

    return pl.pallas_call(
        body,
        out_shape=jax.ShapeDtypeStruct((B_loc, Sq, D), f32),
        in_specs=[pl.BlockSpec(memory_space=pltpu.VMEM)] * 8,
        out_specs=pl.BlockSpec(memory_space=pltpu.VMEM),
        scratch_shapes=[
            pltpu.VMEM((T, D), bf16),
            pltpu.VMEM((T, HD), bf16),
            pltpu.VMEM((T, D), f32),
            pltpu.VMEM((2, D + HD, D), bf16),
            pltpu.SemaphoreType.DMA((2,)),
            pltpu.SemaphoreType.DMA((2,)),
        ],
        compiler_params=pltpu.CompilerParams(collective_id=0),
    )(x, Wq, Wk, Wv, Wo, cosw, sinw, P)
